# baseline (speedup 1.0000x reference)
import jax
import jax.numpy as jnp
from jax import lax
from jax.experimental import pallas as pl
from jax.experimental.pallas import tpu as pltpu


def _make_fused_kernel(H, W, Cin, Cmid):
    HW = H * W

    def body(x_ref, w1_ref, g_ref, be_ref, mu_ref, va_ref, ep_ref, bc_ref,
             w2_ref, b2_ref, out_ref):
        x = x_ref[0].astype(jnp.bfloat16)

        y = lax.dot_general(w1_ref[...], x,
                            (((1,), (1,)), ((), ())),
                            preferred_element_type=jnp.float32)

        col = lax.broadcasted_iota(jnp.int32, (1, HW), 1)
        yy = col // W
        xx = col - yy * W
        row_ok = {-1: yy >= 1, 0: None, 1: yy <= H - 2}
        col_ok = {-1: xx >= 1, 0: None, 1: xx <= W - 2}

        parts = []
        t = 0
        for dy in (-1, 0, 1):
            for dx in (-1, 0, 1):
                s = dy * W + dx
                part = y[t * Cmid:(t + 1) * Cmid]
                if s != 0:
                    part = pltpu.roll(part, (-s) % HW, 1)
                conds = [c for c in (row_ok[dy], col_ok[dx]) if c is not None]
                if conds:
                    valid = conds[0]
                    for c in conds[1:]:
                        valid = jnp.logical_and(valid, c)
                    part = jnp.where(valid, part, 0.0)
                parts.append(part)
                t += 1
        while len(parts) > 1:
            parts = [parts[i] + parts[i + 1] if i + 1 < len(parts)
                     else parts[i] for i in range(0, len(parts), 2)]
        acc = parts[0]

        sc = g_ref[...] * lax.rsqrt(va_ref[...] + ep_ref[0, 0])
        bi = (bc_ref[...] - mu_ref[...]) * sc + be_ref[...]
        s1 = jnp.transpose(sc)
        b1 = jnp.transpose(bi)

        h = jnp.maximum(acc * s1 + b1, 0.0)

        out = jnp.dot(w2_ref[...], h, preferred_element_type=jnp.float32)
        out_ref[...] = (out + b2_ref[...])[None]

    return body


def kernel(x_nchw, w1, b1_conv, gamma, beta, mean, var, eps, w2, b2):
    N, Cin, H, W = x_nchw.shape
    Cmid = w1.shape[0]
    Cout = w2.shape[0]
    HW = H * W

    x_t = jnp.transpose(x_nchw, (0, 2, 3, 1)).reshape(N, HW, Cin)

    w1_k = (jnp.transpose(w1, (2, 3, 0, 1))
            .reshape(9 * Cmid, Cin).astype(jnp.bfloat16))

    g = gamma.reshape(1, Cmid)
    be = beta.reshape(1, Cmid)
    mu = mean.reshape(1, Cmid)
    va = var.reshape(1, Cmid)
    bc = b1_conv.reshape(1, Cmid)
    ep = eps.reshape(1, 1)

    w2_k = w2[:, :, 0, 0].astype(jnp.float32)
    b2_k = b2.reshape(Cout, 1).astype(jnp.float32)

    out_flat = pl.pallas_call(
        _make_fused_kernel(H, W, Cin, Cmid),
        out_shape=jax.ShapeDtypeStruct((N, Cout, HW), jnp.float32),
        grid=(N,),
        in_specs=[
            pl.BlockSpec((1, HW, Cin), lambda n: (n, 0, 0)),
            pl.BlockSpec((9 * Cmid, Cin), lambda n: (0, 0)),
            pl.BlockSpec((1, Cmid), lambda n: (0, 0)),
            pl.BlockSpec((1, Cmid), lambda n: (0, 0)),
            pl.BlockSpec((1, Cmid), lambda n: (0, 0)),
            pl.BlockSpec((1, Cmid), lambda n: (0, 0)),
            pl.BlockSpec((1, 1), lambda n: (0, 0)),
            pl.BlockSpec((1, Cmid), lambda n: (0, 0)),
            pl.BlockSpec((Cout, Cmid), lambda n: (0, 0)),
            pl.BlockSpec((Cout, 1), lambda n: (0, 0)),
        ],
        out_specs=pl.BlockSpec((1, Cout, HW), lambda n: (n, 0, 0)),
        compiler_params=pltpu.CompilerParams(
            dimension_semantics=("parallel",),
            vmem_limit_bytes=64 * 1024 * 1024),
    )(x_t, w1_k, g, be, mu, va, ep, bc, w2_k, b2_k)

    return out_flat.reshape(N, Cout, H, W)


# speedup vs baseline: 1.1506x; 1.1506x over previous
import jax
import jax.numpy as jnp
from jax import lax
from jax.experimental import pallas as pl
from jax.experimental.pallas import tpu as pltpu


def _make_fused_kernel(H, W, Cin, Cmid):
    HW = H * W

    def body(x_ref, w1_ref, s1_ref, b1_ref, w2_ref, b2_ref, out_ref):
        x = x_ref[0].astype(jnp.bfloat16)

        y = lax.dot_general(w1_ref[...], x,
                            (((1,), (1,)), ((), ())),
                            preferred_element_type=jnp.float32)

        col = lax.broadcasted_iota(jnp.int32, (1, HW), 1)
        yy = col // W
        xx = col - yy * W
        row_ok = {-1: yy >= 1, 0: None, 1: yy <= H - 2}
        col_ok = {-1: xx >= 1, 0: None, 1: xx <= W - 2}

        parts = []
        t = 0
        for dy in (-1, 0, 1):
            for dx in (-1, 0, 1):
                s = dy * W + dx
                part = y[t * Cmid:(t + 1) * Cmid]
                if s != 0:
                    part = pltpu.roll(part, (-s) % HW, 1)
                conds = [c for c in (row_ok[dy], col_ok[dx]) if c is not None]
                if conds:
                    valid = conds[0]
                    for c in conds[1:]:
                        valid = jnp.logical_and(valid, c)
                    part = jnp.where(valid, part, 0.0)
                parts.append(part)
                t += 1
        while len(parts) > 1:
            parts = [parts[i] + parts[i + 1] if i + 1 < len(parts)
                     else parts[i] for i in range(0, len(parts), 2)]
        acc = parts[0]

        h = jnp.maximum(acc * s1_ref[...] + b1_ref[...], 0.0)

        out = jnp.dot(w2_ref[...], h, preferred_element_type=jnp.float32)
        out_ref[...] = (out + b2_ref[...])[None]

    return body


def kernel(x_nchw, w1, b1_conv, gamma, beta, mean, var, eps, w2, b2):
    N, Cin, H, W = x_nchw.shape
    Cmid = w1.shape[0]
    Cout = w2.shape[0]
    HW = H * W

    x_t = jnp.transpose(x_nchw, (0, 2, 3, 1)).reshape(N, HW, Cin)

    w1_k = (jnp.transpose(w1, (2, 3, 0, 1))
            .reshape(9 * Cmid, Cin).astype(jnp.bfloat16))

    scale = gamma / jnp.sqrt(var + eps)
    bias = (b1_conv - mean) * scale + beta
    s1 = scale.reshape(Cmid, 1).astype(jnp.float32)
    b1 = bias.reshape(Cmid, 1).astype(jnp.float32)

    w2_k = w2[:, :, 0, 0].astype(jnp.float32)
    b2_k = b2.reshape(Cout, 1).astype(jnp.float32)

    out_flat = pl.pallas_call(
        _make_fused_kernel(H, W, Cin, Cmid),
        out_shape=jax.ShapeDtypeStruct((N, Cout, HW), jnp.float32),
        grid=(N,),
        in_specs=[
            pl.BlockSpec((1, HW, Cin), lambda n: (n, 0, 0)),
            pl.BlockSpec((9 * Cmid, Cin), lambda n: (0, 0)),
            pl.BlockSpec((Cmid, 1), lambda n: (0, 0)),
            pl.BlockSpec((Cmid, 1), lambda n: (0, 0)),
            pl.BlockSpec((Cout, Cmid), lambda n: (0, 0)),
            pl.BlockSpec((Cout, 1), lambda n: (0, 0)),
        ],
        out_specs=pl.BlockSpec((1, Cout, HW), lambda n: (n, 0, 0)),
        compiler_params=pltpu.CompilerParams(
            dimension_semantics=("parallel",),
            vmem_limit_bytes=64 * 1024 * 1024),
    )(x_t, w1_k, s1, b1, w2_k, b2_k)

    return out_flat.reshape(N, Cout, H, W)


# speedup vs baseline: 1.2179x; 1.0585x over previous
import jax
import jax.numpy as jnp
from jax import lax
from jax.experimental import pallas as pl
from jax.experimental.pallas import tpu as pltpu


def _make_fused_kernel(H, W, Cin, Cmid):
    HW = H * W

    def body(x_ref, w1_ref, s1_ref, b1_ref, w2_ref, b2_ref, out_ref):
        x = x_ref[0].astype(jnp.bfloat16)

        zrow = jnp.zeros((W, Cin), jnp.bfloat16)
        up = jnp.concatenate([zrow, x[:-W]], axis=0)
        dn = jnp.concatenate([x[W:], zrow], axis=0)
        xt = jnp.concatenate([up, x, dn], axis=1)

        y = lax.dot_general(w1_ref[...], xt,
                            (((1,), (1,)), ((), ())),
                            preferred_element_type=jnp.float32)

        xx = lax.broadcasted_iota(jnp.int32, (1, HW), 1) % W
        ok_l = xx >= 1
        ok_r = xx <= W - 2

        mid = y[Cmid:2 * Cmid]
        lft = pltpu.roll(y[:Cmid], 1, 1)
        rgt = pltpu.roll(y[2 * Cmid:], HW - 1, 1)
        acc = (mid + jnp.where(ok_l, lft, 0.0)
               + jnp.where(ok_r, rgt, 0.0))

        h = jnp.maximum(acc * s1_ref[...] + b1_ref[...], 0.0)

        out = jnp.dot(w2_ref[...], h, preferred_element_type=jnp.float32)
        out_ref[...] = (out + b2_ref[...])[None]

    return body


def kernel(x_nchw, w1, b1_conv, gamma, beta, mean, var, eps, w2, b2):
    N, Cin, H, W = x_nchw.shape
    Cmid = w1.shape[0]
    Cout = w2.shape[0]
    HW = H * W

    x_t = jnp.transpose(x_nchw, (0, 2, 3, 1)).reshape(N, HW, Cin)

    w1_k = (jnp.transpose(w1, (3, 0, 2, 1))
            .reshape(3 * Cmid, 3 * Cin).astype(jnp.bfloat16))

    scale = gamma / jnp.sqrt(var + eps)
    bias = (b1_conv - mean) * scale + beta
    s1 = scale.reshape(Cmid, 1).astype(jnp.float32)
    b1 = bias.reshape(Cmid, 1).astype(jnp.float32)

    w2_k = w2[:, :, 0, 0].astype(jnp.float32)
    b2_k = b2.reshape(Cout, 1).astype(jnp.float32)

    out = pl.pallas_call(
        _make_fused_kernel(H, W, Cin, Cmid),
        out_shape=jax.ShapeDtypeStruct((N, Cout, HW), jnp.float32),
        grid=(N,),
        in_specs=[
            pl.BlockSpec((1, HW, Cin), lambda n: (n, 0, 0)),
            pl.BlockSpec((3 * Cmid, 3 * Cin), lambda n: (0, 0)),
            pl.BlockSpec((Cmid, 1), lambda n: (0, 0)),
            pl.BlockSpec((Cmid, 1), lambda n: (0, 0)),
            pl.BlockSpec((Cout, Cmid), lambda n: (0, 0)),
            pl.BlockSpec((Cout, 1), lambda n: (0, 0)),
        ],
        out_specs=pl.BlockSpec((1, Cout, HW), lambda n: (n, 0, 0)),
        compiler_params=pltpu.CompilerParams(
            dimension_semantics=("parallel",),
            vmem_limit_bytes=64 * 1024 * 1024),
    )(x_t, w1_k, s1, b1, w2_k, b2_k)

    return out.reshape(N, Cout, H, W)


# speedup vs baseline: 1.3609x; 1.1174x over previous
import jax
import jax.numpy as jnp
from jax import lax
from jax.experimental import pallas as pl
from jax.experimental.pallas import tpu as pltpu


def _make_fused_kernel(H, W, Cin, Cmid):
    HW = H * W

    def body(x_ref, w1_ref, s1_ref, b1_ref, w2_ref, b2_ref, out_ref):
        x = x_ref[0].astype(jnp.bfloat16)

        zrow = jnp.zeros((W, Cin), jnp.bfloat16)
        up = jnp.concatenate([zrow, x[:-W]], axis=0)
        dn = jnp.concatenate([x[W:], zrow], axis=0)

        dn_dims = (((1,), (1,)), ((), ()))
        y = (lax.dot_general(w1_ref[0].astype(jnp.bfloat16), up, dn_dims,
                             preferred_element_type=jnp.float32)
             + lax.dot_general(w1_ref[1].astype(jnp.bfloat16), x, dn_dims,
                               preferred_element_type=jnp.float32)
             + lax.dot_general(w1_ref[2].astype(jnp.bfloat16), dn, dn_dims,
                               preferred_element_type=jnp.float32))

        xx = lax.broadcasted_iota(jnp.int32, (1, HW), 1) % W
        ok_l = xx >= 1
        ok_r = xx <= W - 2

        mid = y[Cmid:2 * Cmid]
        lft = pltpu.roll(y[:Cmid], 1, 1)
        rgt = pltpu.roll(y[2 * Cmid:], HW - 1, 1)
        acc = (mid + jnp.where(ok_l, lft, 0.0)
               + jnp.where(ok_r, rgt, 0.0))

        s1 = s1_ref[:, :1]
        b1 = b1_ref[:, :1]
        h = jnp.maximum(acc * s1 + b1, 0.0)

        out = jnp.dot(w2_ref[...], h, preferred_element_type=jnp.float32)
        out = out + b2_ref[...]
        out_ref[...] = out.reshape(1, H, W)

    return body


def kernel(x_nchw, w1, b1_conv, gamma, beta, mean, var, eps, w2, b2):
    N, Cin, H, W = x_nchw.shape
    Cmid = w1.shape[0]
    Cout = w2.shape[0]
    HW = H * W

    x_t = jnp.transpose(x_nchw, (0, 2, 3, 1)).reshape(N, HW, Cin)

    w1_k = jnp.transpose(w1, (2, 3, 0, 1)).reshape(3, 3 * Cmid, Cin)

    scale = gamma / jnp.sqrt(var + eps)
    bias = (b1_conv - mean) * scale + beta
    s1 = jnp.broadcast_to(scale.reshape(Cmid, 1), (Cmid, 128))
    b1 = jnp.broadcast_to(bias.reshape(Cmid, 1), (Cmid, 128))

    w2_k = w2[:, :, 0, 0].astype(jnp.float32)
    b2_k = b2.reshape(Cout, 1).astype(jnp.float32)

    out = pl.pallas_call(
        _make_fused_kernel(H, W, Cin, Cmid),
        out_shape=jax.ShapeDtypeStruct((N, H, W), jnp.float32),
        grid=(N,),
        in_specs=[
            pl.BlockSpec((1, HW, Cin), lambda n: (n, 0, 0)),
            pl.BlockSpec((3, 3 * Cmid, Cin), lambda n: (0, 0, 0)),
            pl.BlockSpec((Cmid, 128), lambda n: (0, 0)),
            pl.BlockSpec((Cmid, 128), lambda n: (0, 0)),
            pl.BlockSpec((Cout, Cmid), lambda n: (0, 0)),
            pl.BlockSpec((Cout, 1), lambda n: (0, 0)),
        ],
        out_specs=pl.BlockSpec((1, H, W), lambda n: (n, 0, 0)),
        compiler_params=pltpu.CompilerParams(
            dimension_semantics=("parallel",),
            vmem_limit_bytes=64 * 1024 * 1024),
    )(x_t, w1_k, s1, b1, w2_k, b2_k)

    return out.reshape(N, Cout, H, W)
